# baseline (speedup 1.0000x reference)
import functools

import jax
import jax.numpy as jnp
from jax import lax
from jax.experimental import pallas as pl
from jax.experimental.pallas import tpu as pltpu
from jax.experimental.pallas import tpu_sc as plsc

_K = 256
_N = 8192
_ROWS = 64
_NV = _N // 16
_HSTRIDE = 257
_HSIZE = 16 * 264


def _sc_body(x_hbm, out_hbm, xrow, keys, cbuf, cbuf2, hist, counts):
    wid = lax.axis_index("s") * 2 + lax.axis_index("c")
    iota = lax.iota(jnp.int32, 16)
    lane_base = iota * _HSTRIDE
    ones = jnp.ones((16,), jnp.int32)
    zeros16 = jnp.zeros((16,), jnp.int32)

    def zero_hist():
        @plsc.parallel_loop(0, _HSIZE // 16, unroll=8)
        def _(i):
            hist[pl.ds(i * 16, 16)] = zeros16

    def select(k):
        @plsc.parallel_loop(0, 16, unroll=2)
        def _(g):
            acc = zeros16
            for l in range(16):
                acc = acc + plsc.load_gather(
                    hist, [l * _HSTRIDE + g * 16 + iota]
                )
            counts[pl.ds(g * 16, 16)] = acc

        gtot = zeros16
        for j in range(16):
            gtot = gtot + plsc.load_gather(counts, [iota * 16 + j])

        sfx_g = lax.rev(plsc.cumsum(lax.rev(gtot, (0,))), (0,))
        gsel = jnp.sum((sfx_g >= k).astype(jnp.int32)) - 1
        above_g = jnp.sum(jnp.where(iota == gsel, sfx_g - gtot, 0))

        cv = counts[pl.ds(gsel * 16, 16)]
        sfx_in = lax.rev(plsc.cumsum(lax.rev(cv, (0,))), (0,)) + above_g
        dsel = jnp.sum((sfx_in >= k).astype(jnp.int32)) - 1
        above_d = jnp.sum(jnp.where(iota == dsel, sfx_in - cv, 0))
        nbin = jnp.sum(jnp.where(iota == dsel, cv, 0))
        return gsel * 16 + dsel, above_d, nbin

    for r in range(2):
        row = wid * 2 + r
        pltpu.sync_copy(x_hbm.at[row], xrow)

        zero_hist()

        @plsc.parallel_loop(0, _NV, unroll=8)
        def _(i):
            xv = xrow[pl.ds(i * 16, 16)]
            b = lax.bitcast_convert_type(xv, jnp.int32)
            key = jnp.where(b >= 0, b, b ^ jnp.int32(0x7FFFFFFF))
            keys[pl.ds(i * 16, 16)] = key
            digit = (key >> 24) + 128
            plsc.addupdate_scatter(hist, [lane_base + digit], ones)

        d0, above0, nbin0 = select(_K)
        k1 = _K - above0
        d0v = zeros16 + d0

        @functools.partial(
            plsc.parallel_loop(0, _NV, unroll=8, carry=zeros16)
        )
        def _(i, off):
            kv = keys[pl.ds(i * 16, 16)]
            m = ((kv >> 24) + 128) == d0v
            pos = off + plsc.cumsum(m.astype(jnp.int32)) - 1
            plsc.store_scatter(cbuf, [pos], kv, mask=m)
            return off + plsc.all_reduce_population_count(m)

        zero_hist()
        nv1 = (nbin0 + 15) // 16
        nb0v = zeros16 + nbin0

        @plsc.parallel_loop(0, nv1, unroll=4)
        def _(i):
            kv = cbuf[pl.ds(i * 16, 16)]
            valid = (i * 16 + iota) < nb0v
            digit = (kv >> 16) & jnp.int32(0xFF)
            plsc.addupdate_scatter(
                hist, [lane_base + digit], ones, mask=valid
            )

        d1, above1, nbin1 = select(k1)
        k2 = k1 - above1
        d1v = zeros16 + d1

        @functools.partial(
            plsc.parallel_loop(0, nv1, unroll=4, carry=zeros16)
        )
        def _(i, off):
            kv = cbuf[pl.ds(i * 16, 16)]
            valid = (i * 16 + iota) < nb0v
            m = (((kv >> 16) & jnp.int32(0xFF)) == d1v) & valid
            pos = off + plsc.cumsum(m.astype(jnp.int32)) - 1
            plsc.store_scatter(cbuf2, [pos], kv, mask=m)
            return off + plsc.all_reduce_population_count(m)

        nv2 = (nbin1 + 15) // 16
        k2v = zeros16 + k2
        nb1v = zeros16 + nbin1
        pref0 = zeros16 + (((d0 - 128) << 24) | (d1 << 16))

        def bit_body(bit, pref):
            cand = pref | (jnp.int32(1) << (15 - bit))

            def cnt_body(v, cnt):
                kv = cbuf2[pl.ds(v * 16, 16)]
                m = (kv >= cand) & ((v * 16 + iota) < nb1v)
                return cnt + plsc.all_reduce_population_count(m)

            cnt = plsc.parallel_loop(0, nv2, unroll=1, carry=zeros16)(
                cnt_body
            )
            return jnp.where(cnt >= k2v, cand, pref)

        tk = lax.fori_loop(0, 16, bit_body, pref0)
        tb = jnp.where(tk >= 0, tk, tk ^ jnp.int32(0x7FFFFFFF))
        tf = lax.bitcast_convert_type(tb, jnp.float32)

        @plsc.parallel_loop(0, _NV, unroll=8)
        def _(i):
            xv = xrow[pl.ds(i * 16, 16)]
            xrow[pl.ds(i * 16, 16)] = jnp.where(
                xv >= tf, xv, jnp.zeros((16,), jnp.float32)
            )

        pltpu.sync_copy(xrow, out_hbm.at[row])


@jax.jit
def kernel(x):
    mesh = plsc.VectorSubcoreMesh(core_axis_name="c", subcore_axis_name="s")
    f = functools.partial(
        pl.kernel,
        mesh=mesh,
        out_type=jax.ShapeDtypeStruct((_ROWS, _N), jnp.float32),
        compiler_params=pltpu.CompilerParams(
            needs_layout_passes=False,
            disable_bounds_checks=True,
            disable_semaphore_checks=True,
            skip_device_barrier=True,
        ),
        scratch_types=[
            pltpu.VMEM((_N,), jnp.float32),
            pltpu.VMEM((_N,), jnp.int32),
            pltpu.VMEM((_N,), jnp.int32),
            pltpu.VMEM((_N,), jnp.int32),
            pltpu.VMEM((_HSIZE,), jnp.int32),
            pltpu.VMEM((256,), jnp.int32),
        ],
    )(_sc_body)
    return f(x)


# speedup vs baseline: 1.0306x; 1.0306x over previous
import functools

import jax
import jax.numpy as jnp
from jax import lax
from jax.experimental import pallas as pl
from jax.experimental.pallas import tpu as pltpu
from jax.experimental.pallas import tpu_sc as plsc

_K = 256
_N = 8192
_ROWS = 64
_SC_ROWS = 32
_NV = _N // 16
_HSTRIDE = 257
_HSIZE = 16 * 264


def _sc_body(x_hbm, out_hbm, xrow, keys, cbuf, cbuf2, hist, counts):
    wid = lax.axis_index("s") * 2 + lax.axis_index("c")
    iota = lax.iota(jnp.int32, 16)
    lane_base = iota * _HSTRIDE
    ones = jnp.ones((16,), jnp.int32)
    zeros16 = jnp.zeros((16,), jnp.int32)

    def zero_hist():
        @plsc.parallel_loop(0, _HSIZE // 16, unroll=8)
        def _(i):
            hist[pl.ds(i * 16, 16)] = zeros16

    def select(k):
        @plsc.parallel_loop(0, 16, unroll=2)
        def _(g):
            acc = zeros16
            for l in range(16):
                acc = acc + plsc.load_gather(
                    hist, [l * _HSTRIDE + g * 16 + iota]
                )
            counts[pl.ds(g * 16, 16)] = acc

        gtot = zeros16
        for j in range(16):
            gtot = gtot + plsc.load_gather(counts, [iota * 16 + j])

        sfx_g = lax.rev(plsc.cumsum(lax.rev(gtot, (0,))), (0,))
        gsel = jnp.sum((sfx_g >= k).astype(jnp.int32)) - 1
        above_g = jnp.sum(jnp.where(iota == gsel, sfx_g - gtot, 0))

        cv = counts[pl.ds(gsel * 16, 16)]
        sfx_in = lax.rev(plsc.cumsum(lax.rev(cv, (0,))), (0,)) + above_g
        dsel = jnp.sum((sfx_in >= k).astype(jnp.int32)) - 1
        above_d = jnp.sum(jnp.where(iota == dsel, sfx_in - cv, 0))
        nbin = jnp.sum(jnp.where(iota == dsel, cv, 0))
        return gsel * 16 + dsel, above_d, nbin

    row = wid
    pltpu.sync_copy(x_hbm.at[row], xrow)

    zero_hist()

    @plsc.parallel_loop(0, _NV, unroll=8)
    def _(i):
        xv = xrow[pl.ds(i * 16, 16)]
        b = lax.bitcast_convert_type(xv, jnp.int32)
        key = jnp.where(b >= 0, b, b ^ jnp.int32(0x7FFFFFFF))
        keys[pl.ds(i * 16, 16)] = key
        digit = (key >> 24) + 128
        plsc.addupdate_scatter(hist, [lane_base + digit], ones)

    d0, above0, nbin0 = select(_K)
    k1 = _K - above0
    d0v = zeros16 + d0

    @functools.partial(plsc.parallel_loop(0, _NV, unroll=8, carry=zeros16))
    def _(i, off):
        kv = keys[pl.ds(i * 16, 16)]
        m = ((kv >> 24) + 128) == d0v
        pos = off + plsc.cumsum(m.astype(jnp.int32)) - 1
        plsc.store_scatter(cbuf, [pos], kv, mask=m)
        return off + plsc.all_reduce_population_count(m)

    zero_hist()
    nv1 = (nbin0 + 15) // 16
    nb0v = zeros16 + nbin0

    @plsc.parallel_loop(0, nv1, unroll=4)
    def _(i):
        kv = cbuf[pl.ds(i * 16, 16)]
        valid = (i * 16 + iota) < nb0v
        digit = (kv >> 16) & jnp.int32(0xFF)
        plsc.addupdate_scatter(hist, [lane_base + digit], ones, mask=valid)

    d1, above1, nbin1 = select(k1)
    k2 = k1 - above1
    d1v = zeros16 + d1

    @functools.partial(plsc.parallel_loop(0, nv1, unroll=4, carry=zeros16))
    def _(i, off):
        kv = cbuf[pl.ds(i * 16, 16)]
        valid = (i * 16 + iota) < nb0v
        m = (((kv >> 16) & jnp.int32(0xFF)) == d1v) & valid
        pos = off + plsc.cumsum(m.astype(jnp.int32)) - 1
        plsc.store_scatter(cbuf2, [pos], kv, mask=m)
        return off + plsc.all_reduce_population_count(m)

    nv2 = (nbin1 + 15) // 16
    k2v = zeros16 + k2
    nb1v = zeros16 + nbin1
    pref0 = zeros16 + (((d0 - 128) << 24) | (d1 << 16))

    def bit_body(bit, pref):
        cand = pref | (jnp.int32(1) << (15 - bit))

        def cnt_body(v, cnt):
            kv = cbuf2[pl.ds(v * 16, 16)]
            m = (kv >= cand) & ((v * 16 + iota) < nb1v)
            return cnt + plsc.all_reduce_population_count(m)

        cnt = plsc.parallel_loop(0, nv2, unroll=1, carry=zeros16)(cnt_body)
        return jnp.where(cnt >= k2v, cand, pref)

    tk = lax.fori_loop(0, 16, bit_body, pref0)
    tb = jnp.where(tk >= 0, tk, tk ^ jnp.int32(0x7FFFFFFF))
    tf = lax.bitcast_convert_type(tb, jnp.float32)

    @plsc.parallel_loop(0, _NV, unroll=8)
    def _(i):
        xv = xrow[pl.ds(i * 16, 16)]
        xrow[pl.ds(i * 16, 16)] = jnp.where(
            xv >= tf, xv, jnp.zeros((16,), jnp.float32)
        )

    pltpu.sync_copy(xrow, out_hbm.at[row])


def _tc_body(x_ref, o_ref):
    x = x_ref[...]
    rows = x.shape[0]
    b = jax.lax.bitcast_convert_type(x, jnp.uint32)
    key = jnp.where(b >> 31 == jnp.uint32(1), ~b, b | jnp.uint32(0x80000000))

    def step(i, prefix):
        bit = jnp.uint32(1) << (jnp.uint32(31) - jnp.uint32(i))
        cand = prefix | bit
        cnt = jnp.sum((key >= cand).astype(jnp.int32), axis=-1, keepdims=True)
        return jnp.where(cnt >= _K, cand, prefix)

    prefix = jnp.zeros((rows, 1), jnp.uint32)
    thresh = jax.lax.fori_loop(0, 32, step, prefix)
    o_ref[...] = jnp.where(key >= thresh, x, jnp.zeros_like(x))


@jax.jit
def kernel(x):
    mesh = plsc.VectorSubcoreMesh(core_axis_name="c", subcore_axis_name="s")
    sc_fn = functools.partial(
        pl.kernel,
        mesh=mesh,
        out_type=jax.ShapeDtypeStruct((_SC_ROWS, _N), jnp.float32),
        compiler_params=pltpu.CompilerParams(
            needs_layout_passes=False,
            disable_bounds_checks=True,
            disable_semaphore_checks=True,
        ),
        scratch_types=[
            pltpu.VMEM((_N,), jnp.float32),
            pltpu.VMEM((_N,), jnp.int32),
            pltpu.VMEM((_N,), jnp.int32),
            pltpu.VMEM((_N,), jnp.int32),
            pltpu.VMEM((_HSIZE,), jnp.int32),
            pltpu.VMEM((256,), jnp.int32),
        ],
    )(_sc_body)
    sc_out = sc_fn(x[:_SC_ROWS])
    tc_out = pl.pallas_call(
        _tc_body,
        out_shape=jax.ShapeDtypeStruct((_ROWS - _SC_ROWS, _N), x.dtype),
    )(x[_SC_ROWS:])
    return jnp.concatenate([sc_out, tc_out], axis=0)


# speedup vs baseline: 1.1060x; 1.0732x over previous
import functools

import jax
import jax.numpy as jnp
from jax import lax
from jax.experimental import pallas as pl
from jax.experimental.pallas import tpu as pltpu
from jax.experimental.pallas import tpu_sc as plsc

_K = 256
_N = 8192
_ROWS = 64
_SC_ROWS = 32
_NV = _N // 16
_HSTRIDE = 257
_HSIZE = 16 * 264


def _sc_body(x_hbm, out_hbm, xrow, keys, cbuf, cbuf2, hist, counts):
    wid = lax.axis_index("s") * 2 + lax.axis_index("c")
    iota = lax.iota(jnp.int32, 16)
    lane_base = iota * _HSTRIDE
    ones = jnp.ones((16,), jnp.int32)
    zeros16 = jnp.zeros((16,), jnp.int32)

    def zero_hist():
        @plsc.parallel_loop(0, _HSIZE // 16, unroll=8)
        def _(i):
            hist[pl.ds(i * 16, 16)] = zeros16

    def select(k):
        @plsc.parallel_loop(0, 16, unroll=2)
        def _(g):
            acc = zeros16
            for l in range(16):
                acc = acc + plsc.load_gather(
                    hist, [l * _HSTRIDE + g * 16 + iota]
                )
            counts[pl.ds(g * 16, 16)] = acc

        gtot = zeros16
        for j in range(16):
            gtot = gtot + plsc.load_gather(counts, [iota * 16 + j])

        sfx_g = lax.rev(plsc.cumsum(lax.rev(gtot, (0,))), (0,))
        gsel = jnp.sum((sfx_g >= k).astype(jnp.int32)) - 1
        above_g = jnp.sum(jnp.where(iota == gsel, sfx_g - gtot, 0))

        cv = counts[pl.ds(gsel * 16, 16)]
        sfx_in = lax.rev(plsc.cumsum(lax.rev(cv, (0,))), (0,)) + above_g
        dsel = jnp.sum((sfx_in >= k).astype(jnp.int32)) - 1
        above_d = jnp.sum(jnp.where(iota == dsel, sfx_in - cv, 0))
        nbin = jnp.sum(jnp.where(iota == dsel, cv, 0))
        return gsel * 16 + dsel, above_d, nbin

    row = wid
    pltpu.sync_copy(x_hbm.at[row], xrow)

    zero_hist()

    @plsc.parallel_loop(0, _NV, unroll=8)
    def _(i):
        xv = xrow[pl.ds(i * 16, 16)]
        b = lax.bitcast_convert_type(xv, jnp.int32)
        key = jnp.where(b >= 0, b, b ^ jnp.int32(0x7FFFFFFF))
        keys[pl.ds(i * 16, 16)] = key
        digit = (key >> 24) + 128
        plsc.addupdate_scatter(hist, [lane_base + digit], ones)

    d0, above0, nbin0 = select(_K)
    k1 = _K - above0
    d0v = zeros16 + d0

    @functools.partial(plsc.parallel_loop(0, _NV, unroll=8, carry=zeros16))
    def _(i, off):
        kv = keys[pl.ds(i * 16, 16)]
        m = ((kv >> 24) + 128) == d0v
        pos = off + plsc.cumsum(m.astype(jnp.int32)) - 1
        plsc.store_scatter(cbuf, [pos], kv, mask=m)
        return off + plsc.all_reduce_population_count(m)

    zero_hist()
    nv1 = (nbin0 + 15) // 16
    nb0v = zeros16 + nbin0

    @plsc.parallel_loop(0, nv1, unroll=4)
    def _(i):
        kv = cbuf[pl.ds(i * 16, 16)]
        valid = (i * 16 + iota) < nb0v
        digit = (kv >> 16) & jnp.int32(0xFF)
        plsc.addupdate_scatter(hist, [lane_base + digit], ones, mask=valid)

    d1, above1, nbin1 = select(k1)
    k2 = k1 - above1
    d1v = zeros16 + d1

    @functools.partial(plsc.parallel_loop(0, nv1, unroll=4, carry=zeros16))
    def _(i, off):
        kv = cbuf[pl.ds(i * 16, 16)]
        valid = (i * 16 + iota) < nb0v
        m = (((kv >> 16) & jnp.int32(0xFF)) == d1v) & valid
        pos = off + plsc.cumsum(m.astype(jnp.int32)) - 1
        plsc.store_scatter(cbuf2, [pos], kv, mask=m)
        return off + plsc.all_reduce_population_count(m)

    nv2 = (nbin1 + 15) // 16
    k2v = zeros16 + k2
    nb1v = zeros16 + nbin1
    pref0 = zeros16 + (((d0 - 128) << 24) | (d1 << 16))

    def bit_body(bit, pref):
        cand = pref | (jnp.int32(1) << (15 - bit))

        def cnt_body(v, cnt):
            kv = cbuf2[pl.ds(v * 16, 16)]
            m = (kv >= cand) & ((v * 16 + iota) < nb1v)
            return cnt + plsc.all_reduce_population_count(m)

        cnt = plsc.parallel_loop(0, nv2, unroll=1, carry=zeros16)(cnt_body)
        return jnp.where(cnt >= k2v, cand, pref)

    tk = lax.fori_loop(0, 16, bit_body, pref0)
    tb = jnp.where(tk >= 0, tk, tk ^ jnp.int32(0x7FFFFFFF))
    tf = lax.bitcast_convert_type(tb, jnp.float32)

    @plsc.parallel_loop(0, _NV, unroll=8)
    def _(i):
        xv = xrow[pl.ds(i * 16, 16)]
        xrow[pl.ds(i * 16, 16)] = jnp.where(
            xv >= tf, xv, jnp.zeros((16,), jnp.float32)
        )

    pltpu.sync_copy(xrow, out_hbm.at[row])


def _tc_body(x_ref, o_ref):
    x = x_ref[_SC_ROWS:, :]
    rows = _ROWS - _SC_ROWS
    b = jax.lax.bitcast_convert_type(x, jnp.uint32)
    key = jnp.where(b >> 31 == jnp.uint32(1), ~b, b | jnp.uint32(0x80000000))

    def step(i, prefix):
        bit = jnp.uint32(1) << (jnp.uint32(31) - jnp.uint32(i))
        cand = prefix | bit
        cnt = jnp.sum((key >= cand).astype(jnp.int32), axis=-1, keepdims=True)
        return jnp.where(cnt >= _K, cand, prefix)

    prefix = jnp.zeros((rows, 1), jnp.uint32)
    thresh = jax.lax.fori_loop(0, 32, step, prefix)
    o_ref[...] = jnp.where(key >= thresh, x, jnp.zeros_like(x))


@jax.jit
def kernel(x):
    mesh = plsc.VectorSubcoreMesh(core_axis_name="c", subcore_axis_name="s")
    sc_fn = functools.partial(
        pl.kernel,
        mesh=mesh,
        out_type=jax.ShapeDtypeStruct((_SC_ROWS, _N), jnp.float32),
        compiler_params=pltpu.CompilerParams(
            needs_layout_passes=False,
            disable_bounds_checks=True,
            disable_semaphore_checks=True,
        ),
        scratch_types=[
            pltpu.VMEM((_N,), jnp.float32),
            pltpu.VMEM((_N,), jnp.int32),
            pltpu.VMEM((_N,), jnp.int32),
            pltpu.VMEM((_N,), jnp.int32),
            pltpu.VMEM((_HSIZE,), jnp.int32),
            pltpu.VMEM((256,), jnp.int32),
        ],
    )(_sc_body)
    sc_out = sc_fn(x)
    tc_out = pl.pallas_call(
        _tc_body,
        out_shape=jax.ShapeDtypeStruct((_ROWS - _SC_ROWS, _N), x.dtype),
    )(x)
    return jnp.concatenate([sc_out, tc_out], axis=0)


# speedup vs baseline: 1.1068x; 1.0007x over previous
import functools

import jax
import jax.numpy as jnp
from jax import lax
from jax.experimental import pallas as pl
from jax.experimental.pallas import tpu as pltpu
from jax.experimental.pallas import tpu_sc as plsc

_K = 256
_N = 8192
_ROWS = 64
_SC_ROWS = 32
_NV = _N // 16
_HSTRIDE = 257
_HSIZE = 16 * 264


def _sc_body(x_hbm, out_hbm, xrow, keys, cbuf, cbuf2, hist, counts):
    wid = lax.axis_index("s") * 2 + lax.axis_index("c")
    iota = lax.iota(jnp.int32, 16)
    lane_base = iota * _HSTRIDE
    ones = jnp.ones((16,), jnp.int32)
    zeros16 = jnp.zeros((16,), jnp.int32)

    def zero_hist():
        @plsc.parallel_loop(0, _HSIZE // 16, unroll=8)
        def _(i):
            hist[pl.ds(i * 16, 16)] = zeros16

    def select(k):
        @plsc.parallel_loop(0, 16, unroll=2)
        def _(g):
            acc = zeros16
            for l in range(16):
                acc = acc + plsc.load_gather(
                    hist, [l * _HSTRIDE + g * 16 + iota]
                )
            counts[pl.ds(g * 16, 16)] = acc

        gtot = zeros16
        for j in range(16):
            gtot = gtot + plsc.load_gather(counts, [iota * 16 + j])

        sfx_g = lax.rev(plsc.cumsum(lax.rev(gtot, (0,))), (0,))
        gsel = jnp.sum((sfx_g >= k).astype(jnp.int32)) - 1
        above_g = jnp.sum(jnp.where(iota == gsel, sfx_g - gtot, 0))

        cv = counts[pl.ds(gsel * 16, 16)]
        sfx_in = lax.rev(plsc.cumsum(lax.rev(cv, (0,))), (0,)) + above_g
        dsel = jnp.sum((sfx_in >= k).astype(jnp.int32)) - 1
        above_d = jnp.sum(jnp.where(iota == dsel, sfx_in - cv, 0))
        nbin = jnp.sum(jnp.where(iota == dsel, cv, 0))
        return gsel * 16 + dsel, above_d, nbin

    row = wid
    pltpu.sync_copy(x_hbm.at[row], xrow)

    zero_hist()

    @plsc.parallel_loop(0, _NV, unroll=8)
    def _(i):
        xv = xrow[pl.ds(i * 16, 16)]
        b = lax.bitcast_convert_type(xv, jnp.int32)
        key = jnp.where(b >= 0, b, b ^ jnp.int32(0x7FFFFFFF))
        keys[pl.ds(i * 16, 16)] = key
        digit = (key >> 24) + 128
        plsc.addupdate_scatter(hist, [lane_base + digit], ones)

    d0, above0, nbin0 = select(_K)
    k1 = _K - above0
    d0v = zeros16 + d0

    @functools.partial(plsc.parallel_loop(0, _NV, unroll=8, carry=zeros16))
    def _(i, off):
        kv = keys[pl.ds(i * 16, 16)]
        m = ((kv >> 24) + 128) == d0v
        pos = off + plsc.cumsum(m.astype(jnp.int32)) - 1
        plsc.store_scatter(cbuf, [pos], kv, mask=m)
        return off + plsc.all_reduce_population_count(m)

    zero_hist()
    nv1 = (nbin0 + 15) // 16
    nb0v = zeros16 + nbin0

    @plsc.parallel_loop(0, nv1, unroll=4)
    def _(i):
        kv = cbuf[pl.ds(i * 16, 16)]
        valid = (i * 16 + iota) < nb0v
        digit = (kv >> 16) & jnp.int32(0xFF)
        plsc.addupdate_scatter(hist, [lane_base + digit], ones, mask=valid)

    d1, above1, nbin1 = select(k1)
    k2 = k1 - above1
    d1v = zeros16 + d1

    @functools.partial(plsc.parallel_loop(0, nv1, unroll=4, carry=zeros16))
    def _(i, off):
        kv = cbuf[pl.ds(i * 16, 16)]
        valid = (i * 16 + iota) < nb0v
        m = (((kv >> 16) & jnp.int32(0xFF)) == d1v) & valid
        pos = off + plsc.cumsum(m.astype(jnp.int32)) - 1
        plsc.store_scatter(cbuf2, [pos], kv, mask=m)
        return off + plsc.all_reduce_population_count(m)

    nv2 = (nbin1 + 15) // 16
    k2v = zeros16 + k2
    nb1v = zeros16 + nbin1
    pref0 = zeros16 + (((d0 - 128) << 24) | (d1 << 16))

    def bit_body(bit, pref):
        cand = pref | (jnp.int32(1) << (15 - bit))

        def cnt_body(v, cnt):
            kv = cbuf2[pl.ds(v * 16, 16)]
            m = (kv >= cand) & ((v * 16 + iota) < nb1v)
            return cnt + plsc.all_reduce_population_count(m)

        cnt = plsc.parallel_loop(0, nv2, unroll=1, carry=zeros16)(cnt_body)
        return jnp.where(cnt >= k2v, cand, pref)

    tk = lax.fori_loop(0, 16, bit_body, pref0)
    tb = jnp.where(tk >= 0, tk, tk ^ jnp.int32(0x7FFFFFFF))
    tf = lax.bitcast_convert_type(tb, jnp.float32)

    @plsc.parallel_loop(0, _NV, unroll=8)
    def _(i):
        xv = xrow[pl.ds(i * 16, 16)]
        xrow[pl.ds(i * 16, 16)] = jnp.where(
            xv >= tf, xv, jnp.zeros((16,), jnp.float32)
        )

    pltpu.sync_copy(xrow, out_hbm.at[row])


def _tc_body(x_ref, o_ref):
    x = x_ref[...]
    rows = _ROWS - _SC_ROWS
    b = jax.lax.bitcast_convert_type(x, jnp.uint32)
    key = jnp.where(b >> 31 == jnp.uint32(1), ~b, b | jnp.uint32(0x80000000))

    def step(i, prefix):
        bit = jnp.uint32(1) << (jnp.uint32(31) - jnp.uint32(i))
        cand = prefix | bit
        cnt = jnp.sum((key >= cand).astype(jnp.int32), axis=-1, keepdims=True)
        return jnp.where(cnt >= _K, cand, prefix)

    prefix = jnp.zeros((rows, 1), jnp.uint32)
    thresh = jax.lax.fori_loop(0, 32, step, prefix)
    o_ref[...] = jnp.where(key >= thresh, x, jnp.zeros_like(x))


@jax.jit
def kernel(x):
    mesh = plsc.VectorSubcoreMesh(core_axis_name="c", subcore_axis_name="s")
    sc_fn = functools.partial(
        pl.kernel,
        mesh=mesh,
        out_type=jax.ShapeDtypeStruct((_SC_ROWS, _N), jnp.float32),
        compiler_params=pltpu.CompilerParams(
            needs_layout_passes=False,
            disable_bounds_checks=True,
            disable_semaphore_checks=True,
        ),
        scratch_types=[
            pltpu.VMEM((_N,), jnp.float32),
            pltpu.VMEM((_N,), jnp.int32),
            pltpu.VMEM((_N,), jnp.int32),
            pltpu.VMEM((_N,), jnp.int32),
            pltpu.VMEM((_HSIZE,), jnp.int32),
            pltpu.VMEM((256,), jnp.int32),
        ],
    )(_sc_body)
    sc_out = sc_fn(x)
    tc_out = pl.pallas_call(
        _tc_body,
        grid=(1,),
        in_specs=[
            pl.BlockSpec((_ROWS - _SC_ROWS, _N), lambda i: (1, 0))
        ],
        out_specs=pl.BlockSpec((_ROWS - _SC_ROWS, _N), lambda i: (0, 0)),
        out_shape=jax.ShapeDtypeStruct((_ROWS - _SC_ROWS, _N), x.dtype),
    )(x)
    return jnp.concatenate([sc_out, tc_out], axis=0)


# speedup vs baseline: 1.1099x; 1.0028x over previous
import functools

import jax
import jax.numpy as jnp
from jax import lax
from jax.experimental import pallas as pl
from jax.experimental.pallas import tpu as pltpu
from jax.experimental.pallas import tpu_sc as plsc

_K = 256
_N = 8192
_ROWS = 64
_SC_ROWS = 32
_NV = _N // 16
_HSTRIDE = 257
_HSIZE = 16 * 264


def _sc_body(x_hbm, out_hbm, xrow, keys, cbuf, cbuf2, hist, counts):
    wid = lax.axis_index("s") * 2 + lax.axis_index("c")
    iota = lax.iota(jnp.int32, 16)
    lane_base = iota * _HSTRIDE
    ones = jnp.ones((16,), jnp.int32)
    zeros16 = jnp.zeros((16,), jnp.int32)

    def zero_hist():
        @plsc.parallel_loop(0, _HSIZE // 16, unroll=8)
        def _(i):
            hist[pl.ds(i * 16, 16)] = zeros16

    def select(k):
        @plsc.parallel_loop(0, 16, unroll=2)
        def _(g):
            acc = zeros16
            for l in range(16):
                acc = acc + plsc.load_gather(
                    hist, [l * _HSTRIDE + g * 16 + iota]
                )
            counts[pl.ds(g * 16, 16)] = acc

        gtot = zeros16
        for j in range(16):
            gtot = gtot + plsc.load_gather(counts, [iota * 16 + j])

        sfx_g = lax.rev(plsc.cumsum(lax.rev(gtot, (0,))), (0,))
        gsel = jnp.sum((sfx_g >= k).astype(jnp.int32)) - 1
        above_g = jnp.sum(jnp.where(iota == gsel, sfx_g - gtot, 0))

        cv = counts[pl.ds(gsel * 16, 16)]
        sfx_in = lax.rev(plsc.cumsum(lax.rev(cv, (0,))), (0,)) + above_g
        dsel = jnp.sum((sfx_in >= k).astype(jnp.int32)) - 1
        above_d = jnp.sum(jnp.where(iota == dsel, sfx_in - cv, 0))
        nbin = jnp.sum(jnp.where(iota == dsel, cv, 0))
        return gsel * 16 + dsel, above_d, nbin

    row = wid
    pltpu.sync_copy(x_hbm.at[row], xrow)

    zero_hist()

    @plsc.parallel_loop(0, _NV, unroll=8)
    def _(i):
        xv = xrow[pl.ds(i * 16, 16)]
        b = lax.bitcast_convert_type(xv, jnp.int32)
        key = jnp.where(b >= 0, b, b ^ jnp.int32(0x7FFFFFFF))
        keys[pl.ds(i * 16, 16)] = key
        digit = (key >> 24) + 128
        plsc.addupdate_scatter(hist, [lane_base + digit], ones)

    d0, above0, nbin0 = select(_K)
    k1 = _K - above0
    d0v = zeros16 + d0

    @functools.partial(plsc.parallel_loop(0, _NV, unroll=8, carry=zeros16))
    def _(i, off):
        kv = keys[pl.ds(i * 16, 16)]
        m = ((kv >> 24) + 128) == d0v
        pos = off + plsc.cumsum(m.astype(jnp.int32)) - 1
        plsc.store_scatter(cbuf, [pos], kv, mask=m)
        return off + plsc.all_reduce_population_count(m)

    zero_hist()
    nv1 = (nbin0 + 15) // 16
    nb0v = zeros16 + nbin0

    @plsc.parallel_loop(0, nv1, unroll=4)
    def _(i):
        kv = cbuf[pl.ds(i * 16, 16)]
        valid = (i * 16 + iota) < nb0v
        digit = (kv >> 16) & jnp.int32(0xFF)
        plsc.addupdate_scatter(hist, [lane_base + digit], ones, mask=valid)

    d1, above1, nbin1 = select(k1)
    k2 = k1 - above1
    d1v = zeros16 + d1

    @functools.partial(plsc.parallel_loop(0, nv1, unroll=4, carry=zeros16))
    def _(i, off):
        kv = cbuf[pl.ds(i * 16, 16)]
        valid = (i * 16 + iota) < nb0v
        m = (((kv >> 16) & jnp.int32(0xFF)) == d1v) & valid
        pos = off + plsc.cumsum(m.astype(jnp.int32)) - 1
        plsc.store_scatter(cbuf2, [pos], kv, mask=m)
        return off + plsc.all_reduce_population_count(m)

    nv2 = (nbin1 + 15) // 16
    k2v = zeros16 + k2
    nb1v = zeros16 + nbin1
    pref0 = zeros16 + (((d0 - 128) << 24) | (d1 << 16))

    def bit_body(bit, pref):
        cand = pref | (jnp.int32(1) << (15 - bit))

        def cnt_body(v, cnt):
            kv = cbuf2[pl.ds(v * 16, 16)]
            m = (kv >= cand) & ((v * 16 + iota) < nb1v)
            return cnt + plsc.all_reduce_population_count(m)

        cnt = plsc.parallel_loop(0, nv2, unroll=1, carry=zeros16)(cnt_body)
        return jnp.where(cnt >= k2v, cand, pref)

    tk = lax.fori_loop(0, 16, bit_body, pref0)
    tb = jnp.where(tk >= 0, tk, tk ^ jnp.int32(0x7FFFFFFF))
    tf = lax.bitcast_convert_type(tb, jnp.float32)

    @plsc.parallel_loop(0, _NV, unroll=8)
    def _(i):
        xv = xrow[pl.ds(i * 16, 16)]
        xrow[pl.ds(i * 16, 16)] = jnp.where(
            xv >= tf, xv, jnp.zeros((16,), jnp.float32)
        )

    pltpu.sync_copy(xrow, out_hbm.at[row])


def _tc_body(x_ref, o_ref):
    x = x_ref[...]
    rows = _ROWS - _SC_ROWS
    b = jax.lax.bitcast_convert_type(x, jnp.uint32)
    key = jnp.where(b >> 31 == jnp.uint32(1), ~b, b | jnp.uint32(0x80000000))

    def step(i, prefix):
        bit = jnp.uint32(1) << (jnp.uint32(31) - jnp.uint32(i))
        cand = prefix | bit
        cnt = jnp.sum((key >= cand).astype(jnp.int32), axis=-1, keepdims=True)
        return jnp.where(cnt >= _K, cand, prefix)

    prefix = jnp.zeros((rows, 1), jnp.uint32)
    thresh = jax.lax.fori_loop(0, 32, step, prefix)
    tb = jnp.where(thresh >> 31 == jnp.uint32(1), thresh ^ jnp.uint32(0x80000000), ~thresh)
    tf = jax.lax.bitcast_convert_type(tb, jnp.float32)
    o_ref[...] = jnp.where(x >= tf, x, jnp.zeros_like(x))


@jax.jit
def kernel(x):
    mesh = plsc.VectorSubcoreMesh(core_axis_name="c", subcore_axis_name="s")
    sc_fn = functools.partial(
        pl.kernel,
        mesh=mesh,
        out_type=jax.ShapeDtypeStruct((_SC_ROWS, _N), jnp.float32),
        compiler_params=pltpu.CompilerParams(
            needs_layout_passes=False,
            disable_bounds_checks=True,
            disable_semaphore_checks=True,
        ),
        scratch_types=[
            pltpu.VMEM((_N,), jnp.float32),
            pltpu.VMEM((_N,), jnp.int32),
            pltpu.VMEM((_N,), jnp.int32),
            pltpu.VMEM((_N,), jnp.int32),
            pltpu.VMEM((_HSIZE,), jnp.int32),
            pltpu.VMEM((256,), jnp.int32),
        ],
    )(_sc_body)
    sc_out = sc_fn(x)
    tc_out = pl.pallas_call(
        _tc_body,
        grid=(1,),
        in_specs=[
            pl.BlockSpec((_ROWS - _SC_ROWS, _N), lambda i: (1, 0))
        ],
        out_specs=pl.BlockSpec((_ROWS - _SC_ROWS, _N), lambda i: (0, 0)),
        out_shape=jax.ShapeDtypeStruct((_ROWS - _SC_ROWS, _N), x.dtype),
    )(x)
    return jnp.concatenate([sc_out, tc_out], axis=0)
